# Initial kernel scaffold; baseline (speedup 1.0000x reference)
import jax
import jax.numpy as jnp
from jax.experimental import pallas as pl


def kernel(x, concat_data, edge_index, edge_weight, w1, b1, g1, bt1, w2, b2, g2, bt2):
    raise NotImplementedError("write your pallas kernel here")


# speedup vs baseline: 1.2568x; 1.2568x over previous
import functools

import jax
import jax.numpy as jnp
from jax import lax
from jax.experimental import pallas as pl
from jax.experimental.pallas import tpu as pltpu
from jax.experimental.pallas import tpu_sc as plsc

F = 128
NT = 16
NE = 64
RING = 4
NQ = 4
VQ = 2500
VQP = 2560
CAPP = 2816
CAPQ = CAPP + 16


def _make_sc_prep(V, E):
    EPT = E // NT
    mesh = plsc.VectorSubcoreMesh(core_axis_name="c", subcore_axis_name="s")
    i32 = jnp.int32

    @functools.partial(
        pl.kernel,
        out_type=[
            jax.ShapeDtypeStruct((NT, NQ, 1, CAPP), i32),
            jax.ShapeDtypeStruct((NT, NQ, 1, CAPP), i32),
            jax.ShapeDtypeStruct((NT, NQ, 1, CAPP), jnp.float32),
        ],
        mesh=mesh,
        compiler_params=pltpu.CompilerParams(needs_layout_passes=False),
        scratch_types=[
            pltpu.VMEM((EPT,), i32),
            pltpu.VMEM((EPT,), i32),
            pltpu.VMEM((EPT,), jnp.float32),
            [pltpu.VMEM((CAPQ,), i32) for _ in range(NQ)],
            [pltpu.VMEM((CAPQ,), i32) for _ in range(NQ)],
            [pltpu.VMEM((CAPQ,), jnp.float32) for _ in range(NQ)],
        ],
    )
    def prep(idxh, dsth, wh, pidx, pdst, pw, idx_v, dst_v, w_v, io, do, wo):
        cid = lax.axis_index("c")
        sid = lax.axis_index("s")

        @pl.when(cid == 0)
        def _():
            base_e = sid * EPT
            pltpu.sync_copy(idxh.at[pl.ds(base_e, EPT)], idx_v)
            pltpu.sync_copy(dsth.at[pl.ds(base_e, EPT)], dst_v)
            pltpu.sync_copy(wh.at[pl.ds(base_e, EPT)], w_v)

            zi = jnp.zeros((16,), i32)
            zf = jnp.zeros((16,), jnp.float32)
            pr = jnp.full((16,), VQ, i32)

            def fill(o, _):
                sl = pl.ds(o * 16, 16)
                for q in range(NQ):
                    io[q][sl] = zi
                    do[q][sl] = pr
                    wo[q][sl] = zf
                return 0

            lax.fori_loop(0, CAPQ // 16, fill, 0)

            def part(g, pos):
                sl = pl.ds(g * 16, 16)
                s = idx_v[sl]
                d = dst_v[sl]
                w = w_v[sl]
                newpos = []
                for q in range(NQ):
                    mq = (d >= q * VQ) & (d < (q + 1) * VQ)
                    pq = pos[q]
                    plsc.store_compressed(io[q].at[pl.ds(pq, 16)], s, mask=mq)
                    plsc.store_compressed(do[q].at[pl.ds(pq, 16)], d - q * VQ,
                                          mask=mq)
                    plsc.store_compressed(wo[q].at[pl.ds(pq, 16)], w, mask=mq)
                    newpos.append(pq + jnp.sum(mq.astype(i32)))
                return tuple(newpos)

            lax.fori_loop(0, EPT // 16, part, (0,) * NQ)

            for q in range(NQ):
                pltpu.sync_copy(io[q].at[pl.ds(0, CAPP)], pidx.at[sid, q, 0])
                pltpu.sync_copy(do[q].at[pl.ds(0, CAPP)], pdst.at[sid, q, 0])
                pltpu.sync_copy(wo[q].at[pl.ds(0, CAPP)], pw.at[sid, q, 0])

    return prep


def _make_sc_spmm(V, E):
    NCH = CAPP // NE
    NOUT = NCH // RING
    RPT = VQP // NT
    mesh = plsc.VectorSubcoreMesh(core_axis_name="c", subcore_axis_name="s")

    @functools.partial(
        pl.kernel,
        out_type=jax.ShapeDtypeStruct((4, NQ, VQP, F), jnp.float32),
        mesh=mesh,
        compiler_params=pltpu.CompilerParams(needs_layout_passes=False),
        scratch_types=[
            pltpu.VMEM((CAPP,), jnp.int32),
            pltpu.VMEM((CAPP,), jnp.float32),
            pltpu.VMEM((CAPP,), jnp.int32),
            pltpu.VMEM((RING, NE), jnp.int32),
            [pltpu.VMEM((NE, F), jnp.float32) for _ in range(RING)],
            [pltpu.VMEM((NE, F), jnp.float32) for _ in range(RING)],
            pltpu.VMEM_SHARED((VQP, F), jnp.float32),
            pltpu.SemaphoreType.DMA((RING,)),
            pltpu.SemaphoreType.DMA((RING,)),
        ],
    )
    def spmm(tbl, pidx, pdst, pw, zeros_h, out, idx_v, w_v, dst_v, dstc,
             grows, srows, acc, gsem, ssem):
        cid = lax.axis_index("c")
        sid = lax.axis_index("s")

        def gather(c, b):
            pltpu.async_copy(tbl.at[idx_v.at[pl.ds(c * NE, NE)]], grows[b],
                             gsem.at[b])

        def gather_wait(b):
            pltpu.make_async_copy(tbl.at[idx_v.at[pl.ds(0, NE)]], grows[b],
                                  gsem.at[b]).wait()

        def scatter(c, b):
            for q in range(NE // 16):
                dstc[b, pl.ds(q * 16, 16)] = dst_v[pl.ds(c * NE + q * 16, 16)]
            pltpu.async_copy(srows[b], acc.at[dstc.at[b]], ssem.at[b],
                             add=True)

        def scatter_wait(b):
            pltpu.make_async_copy(srows[b], acc.at[dstc.at[0]],
                                  ssem.at[b]).wait()

        def scale(c, b):
            def grp(g, __):
                wg = w_v[pl.ds(c * NE + g * 16, 16)]
                for i in range(16):
                    bc = lax.gather(
                        wg, jnp.full((16, 1), i, jnp.int32),
                        lax.GatherDimensionNumbers(
                            offset_dims=(), collapsed_slice_dims=(0,),
                            start_index_map=(0,)),
                        slice_sizes=(1,),
                        mode=lax.GatherScatterMode.PROMISE_IN_BOUNDS)
                    e = g * 16 + i
                    for j in range(F // 16):
                        sl = pl.ds(j * 16, 16)
                        srows[b][e, sl] = grows[b][e, sl] * bc
                return 0

            lax.fori_loop(0, NE // 16, grp, 0)

        def quarter_body(qq, _):
            pltpu.sync_copy(pidx.at[sid, qq, 0], idx_v)
            pltpu.sync_copy(pw.at[sid, qq, 0], w_v)
            pltpu.sync_copy(pdst.at[sid, qq, 0], dst_v)

            def batch_body(k, __):
                bidx = cid * 2 + k
                off = jnp.where(k == 0, cid * 2 * V, V)

                def add_off(o, ___):
                    for u in range(4):
                        s = o * 64 + u * 16
                        idx_v[pl.ds(s, 16)] = idx_v[pl.ds(s, 16)] + off
                    return 0

                lax.fori_loop(0, CAPP // 64, add_off, 0)

                pltpu.sync_copy(zeros_h, acc.at[pl.ds(sid * RPT, RPT)])
                plsc.subcore_barrier()

                for b in range(RING):
                    gather(b, b)

                def outer(o, __):
                    for b in range(RING):
                        c = o * RING + b
                        gather_wait(b)

                        @pl.when(o >= 1)
                        def drain_prev():
                            scatter_wait(b)

                        scale(c, b)
                        scatter(c, b)

                        @pl.when(o < NOUT - 1)
                        def refill():
                            gather((o + 1) * RING + b, b)
                    return 0

                lax.fori_loop(0, NOUT, outer, 0)
                for b in range(RING):
                    scatter_wait(b)
                plsc.subcore_barrier()
                pltpu.sync_copy(acc.at[pl.ds(sid * RPT, RPT)],
                                out.at[bidx, qq, pl.ds(sid * RPT, RPT)])
                plsc.subcore_barrier()
                return 0

            lax.fori_loop(0, 2, batch_body, 0)
            return 0

        lax.fori_loop(0, NQ, quarter_body, 0)

    return spmm


_BLK = 4000


def _k_up_matmul(x2d, wcat, n_rows_out):
    n = wcat.shape[1]
    grid = n_rows_out // _BLK

    def body(x_ref, w_ref, o_ref):
        xb = jnp.repeat(x_ref[...], 4, axis=0)
        o_ref[...] = jnp.dot(xb, w_ref[...],
                             preferred_element_type=jnp.float32)

    return pl.pallas_call(
        body,
        grid=(grid,),
        in_specs=[
            pl.BlockSpec((_BLK // 4, F), lambda g: (g, 0)),
            pl.BlockSpec(wcat.shape, lambda g: (0, 0)),
        ],
        out_specs=pl.BlockSpec((_BLK, n), lambda g: (g, 0)),
        out_shape=jax.ShapeDtypeStruct((n_rows_out, n), jnp.float32),
    )(x2d, wcat)


def _k_axpy2(a2d, s2d):
    R = a2d.shape[0]

    def body(a_ref, s_ref, o_ref):
        o_ref[...] = a_ref[...] + 2.0 * s_ref[...]

    return pl.pallas_call(
        body,
        grid=(R // _BLK,),
        in_specs=[pl.BlockSpec((_BLK, F), lambda g: (g, 0))] * 2,
        out_specs=pl.BlockSpec((_BLK, F), lambda g: (g, 0)),
        out_shape=jax.ShapeDtypeStruct((R, F), jnp.float32),
    )(a2d, s2d)


def _k_add_stats(base2d, s2d):
    R = base2d.shape[0]

    def body(b_ref, s_ref, o_ref, st_ref):
        t = b_ref[...] + s_ref[...]
        o_ref[...] = t

        @pl.when(pl.program_id(0) == 0)
        def _():
            st_ref[...] = jnp.zeros_like(st_ref)

        st_ref[pl.ds(0, 1), :] += jnp.sum(t, axis=0, keepdims=True)
        st_ref[pl.ds(1, 1), :] += jnp.sum(t * t, axis=0, keepdims=True)

    return pl.pallas_call(
        body,
        grid=(R // _BLK,),
        in_specs=[pl.BlockSpec((_BLK, F), lambda g: (g, 0))] * 2,
        out_specs=[
            pl.BlockSpec((_BLK, F), lambda g: (g, 0)),
            pl.BlockSpec((8, F), lambda g: (0, 0)),
        ],
        out_shape=[
            jax.ShapeDtypeStruct((R, F), jnp.float32),
            jax.ShapeDtypeStruct((8, F), jnp.float32),
        ],
    )(base2d, s2d)


def _bn_coeffs(st_ref, g_ref, bt_ref, n_rows):
    mu = st_ref[pl.ds(0, 1), :] / n_rows
    var = st_ref[pl.ds(1, 1), :] / n_rows - mu * mu
    inv = lax.rsqrt(var + 1e-5) * g_ref[pl.ds(0, 1), :]
    return inv, bt_ref[pl.ds(0, 1), :] - mu * inv


def _k_bn_dualmm(h2d, st, skip2d, whcat, wscat, g, bt):
    R = h2d.shape[0]
    n = whcat.shape[1]

    def body(h_ref, st_ref, sk_ref, wh_ref, ws_ref, g_ref, bt_ref, o_ref):
        inv, sh = _bn_coeffs(st_ref, g_ref, bt_ref, float(R))
        hb = jax.nn.relu(h_ref[...] * inv + sh)
        o_ref[...] = (
            jnp.dot(hb, wh_ref[...], preferred_element_type=jnp.float32)
            + jnp.dot(sk_ref[...], ws_ref[...],
                      preferred_element_type=jnp.float32))

    return pl.pallas_call(
        body,
        grid=(R // _BLK,),
        in_specs=[
            pl.BlockSpec((_BLK, F), lambda g_: (g_, 0)),
            pl.BlockSpec((8, F), lambda g_: (0, 0)),
            pl.BlockSpec((_BLK, F), lambda g_: (g_, 0)),
            pl.BlockSpec((F, n), lambda g_: (0, 0)),
            pl.BlockSpec((F, n), lambda g_: (0, 0)),
            pl.BlockSpec((1, F), lambda g_: (0, 0)),
            pl.BlockSpec((1, F), lambda g_: (0, 0)),
        ],
        out_specs=pl.BlockSpec((_BLK, n), lambda g_: (g_, 0)),
        out_shape=jax.ShapeDtypeStruct((R, n), jnp.float32),
    )(h2d, st, skip2d, whcat, wscat, g, bt)


def _k_bn_relu(h2d, st, g, bt):
    R = h2d.shape[0]

    def body(h_ref, st_ref, g_ref, bt_ref, o_ref):
        inv, sh = _bn_coeffs(st_ref, g_ref, bt_ref, float(R))
        o_ref[...] = jax.nn.relu(h_ref[...] * inv + sh)

    return pl.pallas_call(
        body,
        grid=(R // _BLK,),
        in_specs=[
            pl.BlockSpec((_BLK, F), lambda g_: (g_, 0)),
            pl.BlockSpec((8, F), lambda g_: (0, 0)),
            pl.BlockSpec((1, F), lambda g_: (0, 0)),
            pl.BlockSpec((1, F), lambda g_: (0, 0)),
        ],
        out_specs=pl.BlockSpec((_BLK, F), lambda g_: (g_, 0)),
        out_shape=jax.ShapeDtypeStruct((R, F), jnp.float32),
    )(h2d, st, g, bt)


def kernel(x, concat_data, edge_index, edge_weight, w1, b1, g1, bt1,
           w2, b2, g2, bt2):
    B, V_in, _ = x.shape
    V = concat_data.shape[1]
    E = edge_weight.shape[0]
    R = B * V

    dst = edge_index[0]
    src = edge_index[1]

    prep = _make_sc_prep(V, E)
    pidx, pdst, pw = prep(src, dst, edge_weight)

    spmm_p = _make_sc_spmm(V, E)
    zeros_h = jnp.zeros((VQP // NT, F), jnp.float32)

    def spmm(tbl):
        s = spmm_p(tbl, pidx, pdst, pw, zeros_h)
        return s[:, :, :VQ, :].reshape(R, F)

    wcat1 = jnp.concatenate([w1[1], w1[2], w1[0] - w1[2]], axis=1)
    apb1 = _k_up_matmul(x.reshape(B * V_in, F), wcat1, R)
    a1 = apb1[:, :F]
    p1 = apb1[:, F:2 * F]
    base1 = apb1[:, 2 * F:]
    s2 = spmm(p1)
    q1 = _k_axpy2(a1, s2)
    s3 = spmm(q1)
    h1, st1 = _k_add_stats(base1, s3)

    wh = jnp.concatenate([w2[1, :F], w2[2, :F], (w2[0] - w2[2])[:F]], axis=1)
    ws = jnp.concatenate([w2[1, F:], w2[2, F:], (w2[0] - w2[2])[F:]], axis=1)
    skip2d = concat_data.reshape(R, F)
    apb2 = _k_bn_dualmm(h1, st1, skip2d, wh, ws,
                        g1.reshape(1, F), bt1.reshape(1, F))
    a2 = apb2[:, :F]
    p2 = apb2[:, F:2 * F]
    base2 = apb2[:, 2 * F:]
    s2b = spmm(p2)
    q2 = _k_axpy2(a2, s2b)
    s3b = spmm(q2)
    h2, st2 = _k_add_stats(base2, s3b)
    out = _k_bn_relu(h2, st2, g2.reshape(1, F), bt2.reshape(1, F))
    return out.reshape(B, V, F)


# speedup vs baseline: 1.3193x; 1.0497x over previous
import functools

import jax
import jax.numpy as jnp
from jax import lax
from jax.experimental import pallas as pl
from jax.experimental.pallas import tpu as pltpu
from jax.experimental.pallas import tpu_sc as plsc

F = 128
NT = 16
NE = 64
RING = 4
NQ = 4
VQ = 2500
VQP = 2560
CAPP = 2816
CAPQ = CAPP + 16


def _make_sc_prep(V, E):
    EPT = E // NT
    mesh = plsc.VectorSubcoreMesh(core_axis_name="c", subcore_axis_name="s")
    i32 = jnp.int32

    @functools.partial(
        pl.kernel,
        out_type=[
            jax.ShapeDtypeStruct((NT, NQ, 1, CAPP), i32),
            jax.ShapeDtypeStruct((NT, NQ, 1, CAPP), i32),
            jax.ShapeDtypeStruct((NT, NQ, 1, CAPP), jnp.float32),
        ],
        mesh=mesh,
        compiler_params=pltpu.CompilerParams(needs_layout_passes=False),
        scratch_types=[
            pltpu.VMEM((EPT,), i32),
            pltpu.VMEM((EPT,), i32),
            pltpu.VMEM((EPT,), jnp.float32),
            [pltpu.VMEM((CAPQ,), i32) for _ in range(NQ)],
            [pltpu.VMEM((CAPQ,), i32) for _ in range(NQ)],
            [pltpu.VMEM((CAPQ,), jnp.float32) for _ in range(NQ)],
        ],
    )
    def prep(idxh, dsth, wh, pidx, pdst, pw, idx_v, dst_v, w_v, io, do, wo):
        cid = lax.axis_index("c")
        sid = lax.axis_index("s")

        @pl.when(cid == 0)
        def _():
            base_e = sid * EPT
            pltpu.sync_copy(idxh.at[pl.ds(base_e, EPT)], idx_v)
            pltpu.sync_copy(dsth.at[pl.ds(base_e, EPT)], dst_v)
            pltpu.sync_copy(wh.at[pl.ds(base_e, EPT)], w_v)

            zi = jnp.zeros((16,), i32)
            zf = jnp.zeros((16,), jnp.float32)
            pr = jnp.full((16,), VQ, i32)

            def fill(o, _):
                sl = pl.ds(o * 16, 16)
                for q in range(NQ):
                    io[q][sl] = zi
                    do[q][sl] = pr
                    wo[q][sl] = zf
                return 0

            lax.fori_loop(0, CAPQ // 16, fill, 0)

            def part(g, pos):
                sl = pl.ds(g * 16, 16)
                s = idx_v[sl]
                d = dst_v[sl]
                w = w_v[sl]
                newpos = []
                for q in range(NQ):
                    mq = (d >= q * VQ) & (d < (q + 1) * VQ)
                    pq = pos[q]
                    plsc.store_compressed(io[q].at[pl.ds(pq, 16)], s, mask=mq)
                    plsc.store_compressed(do[q].at[pl.ds(pq, 16)], d - q * VQ,
                                          mask=mq)
                    plsc.store_compressed(wo[q].at[pl.ds(pq, 16)], w, mask=mq)
                    newpos.append(pq + jnp.sum(mq.astype(i32)))
                return tuple(newpos)

            lax.fori_loop(0, EPT // 16, part, (0,) * NQ)

            for q in range(NQ):
                pltpu.sync_copy(io[q].at[pl.ds(0, CAPP)], pidx.at[sid, q, 0])
                pltpu.sync_copy(do[q].at[pl.ds(0, CAPP)], pdst.at[sid, q, 0])
                pltpu.sync_copy(wo[q].at[pl.ds(0, CAPP)], pw.at[sid, q, 0])

    return prep


def _make_sc_spmm(V, E):
    NCH = CAPP // NE
    NOUT = NCH // RING
    RPT = VQP // NT
    mesh = plsc.VectorSubcoreMesh(core_axis_name="c", subcore_axis_name="s")

    @functools.partial(
        pl.kernel,
        out_type=jax.ShapeDtypeStruct((4, NQ, VQP, F), jnp.float32),
        mesh=mesh,
        compiler_params=pltpu.CompilerParams(needs_layout_passes=False),
        scratch_types=[
            pltpu.VMEM((CAPP,), jnp.int32),
            pltpu.VMEM((CAPP,), jnp.float32),
            pltpu.VMEM((CAPP,), jnp.int32),
            pltpu.VMEM((RING, NE), jnp.int32),
            [pltpu.VMEM((NE, F), jnp.float32) for _ in range(RING)],
            [pltpu.VMEM((NE, F), jnp.float32) for _ in range(RING)],
            pltpu.VMEM_SHARED((VQP, F), jnp.float32),
            pltpu.SemaphoreType.DMA((RING,)),
            pltpu.SemaphoreType.DMA((RING,)),
        ],
    )
    def spmm(tbl, pidx, pdst, pw, zeros_h, out, idx_v, w_v, dst_v, dstc,
             grows, srows, acc, gsem, ssem):
        cid = lax.axis_index("c")
        sid = lax.axis_index("s")

        def gather(c, b):
            pltpu.async_copy(tbl.at[idx_v.at[pl.ds(c * NE, NE)]], grows[b],
                             gsem.at[b])

        def gather_wait(b):
            pltpu.make_async_copy(tbl.at[idx_v.at[pl.ds(0, NE)]], grows[b],
                                  gsem.at[b]).wait()

        def scatter(c, b):
            for q in range(NE // 16):
                dstc[b, pl.ds(q * 16, 16)] = dst_v[pl.ds(c * NE + q * 16, 16)]
            pltpu.async_copy(srows[b], acc.at[dstc.at[b]], ssem.at[b],
                             add=True)

        def scatter_wait(b):
            pltpu.make_async_copy(srows[b], acc.at[dstc.at[0]],
                                  ssem.at[b]).wait()

        def scale(c, b):
            for g in range(NE // 16):
                wg = w_v[pl.ds(c * NE + g * 16, 16)]
                for i in range(16):
                    bc = lax.gather(
                        wg, jnp.full((16, 1), i, jnp.int32),
                        lax.GatherDimensionNumbers(
                            offset_dims=(), collapsed_slice_dims=(0,),
                            start_index_map=(0,)),
                        slice_sizes=(1,),
                        mode=lax.GatherScatterMode.PROMISE_IN_BOUNDS)
                    e = g * 16 + i
                    for j in range(F // 16):
                        sl = pl.ds(j * 16, 16)
                        srows[b][e, sl] = grows[b][e, sl] * bc

        def quarter_body(qq, _):
            pltpu.sync_copy(pidx.at[sid, qq, 0], idx_v)
            pltpu.sync_copy(pw.at[sid, qq, 0], w_v)
            pltpu.sync_copy(pdst.at[sid, qq, 0], dst_v)

            def batch_body(k, __):
                bidx = cid * 2 + k
                off = jnp.where(k == 0, cid * 2 * V, V)

                def add_off(o, ___):
                    for u in range(4):
                        s = o * 64 + u * 16
                        idx_v[pl.ds(s, 16)] = idx_v[pl.ds(s, 16)] + off
                    return 0

                lax.fori_loop(0, CAPP // 64, add_off, 0)

                pltpu.sync_copy(zeros_h, acc.at[pl.ds(sid * RPT, RPT)])
                plsc.subcore_barrier()

                for b in range(RING):
                    gather(b, b)

                def outer(o, __):
                    for b in range(RING):
                        c = o * RING + b
                        gather_wait(b)

                        @pl.when(o >= 1)
                        def drain_prev():
                            scatter_wait(b)

                        scale(c, b)
                        scatter(c, b)

                        @pl.when(o < NOUT - 1)
                        def refill():
                            gather((o + 1) * RING + b, b)
                    return 0

                lax.fori_loop(0, NOUT, outer, 0)
                for b in range(RING):
                    scatter_wait(b)
                plsc.subcore_barrier()
                pltpu.sync_copy(acc.at[pl.ds(sid * RPT, RPT)],
                                out.at[bidx, qq, pl.ds(sid * RPT, RPT)])
                plsc.subcore_barrier()
                return 0

            lax.fori_loop(0, 2, batch_body, 0)
            return 0

        lax.fori_loop(0, NQ, quarter_body, 0)

    return spmm


_BLK = 4000


def _k_up_matmul(x2d, wcat, n_rows_out):
    n = wcat.shape[1]
    grid = n_rows_out // _BLK

    def body(x_ref, w_ref, o_ref):
        xb = jnp.repeat(x_ref[...], 4, axis=0)
        o_ref[...] = jnp.dot(xb, w_ref[...],
                             preferred_element_type=jnp.float32)

    return pl.pallas_call(
        body,
        grid=(grid,),
        in_specs=[
            pl.BlockSpec((_BLK // 4, F), lambda g: (g, 0)),
            pl.BlockSpec(wcat.shape, lambda g: (0, 0)),
        ],
        out_specs=pl.BlockSpec((_BLK, n), lambda g: (g, 0)),
        out_shape=jax.ShapeDtypeStruct((n_rows_out, n), jnp.float32),
    )(x2d, wcat)


def _k_axpy2(a2d, s2d):
    R = a2d.shape[0]

    def body(a_ref, s_ref, o_ref):
        o_ref[...] = a_ref[...] + 2.0 * s_ref[...]

    return pl.pallas_call(
        body,
        grid=(R // _BLK,),
        in_specs=[pl.BlockSpec((_BLK, F), lambda g: (g, 0))] * 2,
        out_specs=pl.BlockSpec((_BLK, F), lambda g: (g, 0)),
        out_shape=jax.ShapeDtypeStruct((R, F), jnp.float32),
    )(a2d, s2d)


def _k_add_stats(base2d, s2d):
    R = base2d.shape[0]

    def body(b_ref, s_ref, o_ref, st_ref):
        t = b_ref[...] + s_ref[...]
        o_ref[...] = t

        @pl.when(pl.program_id(0) == 0)
        def _():
            st_ref[...] = jnp.zeros_like(st_ref)

        st_ref[pl.ds(0, 1), :] += jnp.sum(t, axis=0, keepdims=True)
        st_ref[pl.ds(1, 1), :] += jnp.sum(t * t, axis=0, keepdims=True)

    return pl.pallas_call(
        body,
        grid=(R // _BLK,),
        in_specs=[pl.BlockSpec((_BLK, F), lambda g: (g, 0))] * 2,
        out_specs=[
            pl.BlockSpec((_BLK, F), lambda g: (g, 0)),
            pl.BlockSpec((8, F), lambda g: (0, 0)),
        ],
        out_shape=[
            jax.ShapeDtypeStruct((R, F), jnp.float32),
            jax.ShapeDtypeStruct((8, F), jnp.float32),
        ],
    )(base2d, s2d)


def _bn_coeffs(st_ref, g_ref, bt_ref, n_rows):
    mu = st_ref[pl.ds(0, 1), :] / n_rows
    var = st_ref[pl.ds(1, 1), :] / n_rows - mu * mu
    inv = lax.rsqrt(var + 1e-5) * g_ref[pl.ds(0, 1), :]
    return inv, bt_ref[pl.ds(0, 1), :] - mu * inv


def _k_bn_dualmm(h2d, st, skip2d, whcat, wscat, g, bt):
    R = h2d.shape[0]
    n = whcat.shape[1]

    def body(h_ref, st_ref, sk_ref, wh_ref, ws_ref, g_ref, bt_ref, o_ref):
        inv, sh = _bn_coeffs(st_ref, g_ref, bt_ref, float(R))
        hb = jax.nn.relu(h_ref[...] * inv + sh)
        o_ref[...] = (
            jnp.dot(hb, wh_ref[...], preferred_element_type=jnp.float32)
            + jnp.dot(sk_ref[...], ws_ref[...],
                      preferred_element_type=jnp.float32))

    return pl.pallas_call(
        body,
        grid=(R // _BLK,),
        in_specs=[
            pl.BlockSpec((_BLK, F), lambda g_: (g_, 0)),
            pl.BlockSpec((8, F), lambda g_: (0, 0)),
            pl.BlockSpec((_BLK, F), lambda g_: (g_, 0)),
            pl.BlockSpec((F, n), lambda g_: (0, 0)),
            pl.BlockSpec((F, n), lambda g_: (0, 0)),
            pl.BlockSpec((1, F), lambda g_: (0, 0)),
            pl.BlockSpec((1, F), lambda g_: (0, 0)),
        ],
        out_specs=pl.BlockSpec((_BLK, n), lambda g_: (g_, 0)),
        out_shape=jax.ShapeDtypeStruct((R, n), jnp.float32),
    )(h2d, st, skip2d, whcat, wscat, g, bt)


def _k_bn_relu(h2d, st, g, bt):
    R = h2d.shape[0]

    def body(h_ref, st_ref, g_ref, bt_ref, o_ref):
        inv, sh = _bn_coeffs(st_ref, g_ref, bt_ref, float(R))
        o_ref[...] = jax.nn.relu(h_ref[...] * inv + sh)

    return pl.pallas_call(
        body,
        grid=(R // _BLK,),
        in_specs=[
            pl.BlockSpec((_BLK, F), lambda g_: (g_, 0)),
            pl.BlockSpec((8, F), lambda g_: (0, 0)),
            pl.BlockSpec((1, F), lambda g_: (0, 0)),
            pl.BlockSpec((1, F), lambda g_: (0, 0)),
        ],
        out_specs=pl.BlockSpec((_BLK, F), lambda g_: (g_, 0)),
        out_shape=jax.ShapeDtypeStruct((R, F), jnp.float32),
    )(h2d, st, g, bt)


def kernel(x, concat_data, edge_index, edge_weight, w1, b1, g1, bt1,
           w2, b2, g2, bt2):
    B, V_in, _ = x.shape
    V = concat_data.shape[1]
    E = edge_weight.shape[0]
    R = B * V

    dst = edge_index[0]
    src = edge_index[1]

    prep = _make_sc_prep(V, E)
    pidx, pdst, pw = prep(src, dst, edge_weight)

    spmm_p = _make_sc_spmm(V, E)
    zeros_h = jnp.zeros((VQP // NT, F), jnp.float32)

    def spmm(tbl):
        s = spmm_p(tbl, pidx, pdst, pw, zeros_h)
        return s[:, :, :VQ, :].reshape(R, F)

    wcat1 = jnp.concatenate([w1[1], w1[2], w1[0] - w1[2]], axis=1)
    apb1 = _k_up_matmul(x.reshape(B * V_in, F), wcat1, R)
    a1 = apb1[:, :F]
    p1 = apb1[:, F:2 * F]
    base1 = apb1[:, 2 * F:]
    s2 = spmm(p1)
    q1 = _k_axpy2(a1, s2)
    s3 = spmm(q1)
    h1, st1 = _k_add_stats(base1, s3)

    wh = jnp.concatenate([w2[1, :F], w2[2, :F], (w2[0] - w2[2])[:F]], axis=1)
    ws = jnp.concatenate([w2[1, F:], w2[2, F:], (w2[0] - w2[2])[F:]], axis=1)
    skip2d = concat_data.reshape(R, F)
    apb2 = _k_bn_dualmm(h1, st1, skip2d, wh, ws,
                        g1.reshape(1, F), bt1.reshape(1, F))
    a2 = apb2[:, :F]
    p2 = apb2[:, F:2 * F]
    base2 = apb2[:, 2 * F:]
    s2b = spmm(p2)
    q2 = _k_axpy2(a2, s2b)
    s3b = spmm(q2)
    h2, st2 = _k_add_stats(base2, s3b)
    out = _k_bn_relu(h2, st2, g2.reshape(1, F), bt2.reshape(1, F))
    return out.reshape(B, V, F)


# speedup vs baseline: 2.0682x; 1.5677x over previous
import functools

import jax
import jax.numpy as jnp
from jax import lax
from jax.experimental import pallas as pl
from jax.experimental.pallas import tpu as pltpu
from jax.experimental.pallas import tpu_sc as plsc

F = 128
NT = 16
NE = 16
RING = 8
NQ = 2
VQ = 5000
VQP = 5120
CAPP = 5376
CAPQ = CAPP + 16


def _make_sc_prep(V, E):
    EPT = E // NT
    mesh = plsc.VectorSubcoreMesh(core_axis_name="c", subcore_axis_name="s")
    i32 = jnp.int32

    @functools.partial(
        pl.kernel,
        out_type=[
            jax.ShapeDtypeStruct((NT, NQ, 1, CAPP), i32),
            jax.ShapeDtypeStruct((NT, NQ, 1, CAPP), i32),
            jax.ShapeDtypeStruct((NT, NQ, 1, CAPP), jnp.float32),
        ],
        mesh=mesh,
        compiler_params=pltpu.CompilerParams(needs_layout_passes=False),
        scratch_types=[
            pltpu.VMEM((EPT,), i32),
            pltpu.VMEM((EPT,), i32),
            pltpu.VMEM((EPT,), jnp.float32),
            [pltpu.VMEM((CAPQ,), i32) for _ in range(NQ)],
            [pltpu.VMEM((CAPQ,), i32) for _ in range(NQ)],
            [pltpu.VMEM((CAPQ,), jnp.float32) for _ in range(NQ)],
        ],
    )
    def prep(idxh, dsth, wh, pidx, pdst, pw, idx_v, dst_v, w_v, io, do, wo):
        cid = lax.axis_index("c")
        sid = lax.axis_index("s")

        @pl.when(cid == 0)
        def _():
            base_e = sid * EPT
            pltpu.sync_copy(idxh.at[pl.ds(base_e, EPT)], idx_v)
            pltpu.sync_copy(dsth.at[pl.ds(base_e, EPT)], dst_v)
            pltpu.sync_copy(wh.at[pl.ds(base_e, EPT)], w_v)

            zi = jnp.zeros((16,), i32)
            zf = jnp.zeros((16,), jnp.float32)
            pr = jnp.full((16,), VQ, i32)

            def fill(o, _):
                sl = pl.ds(o * 16, 16)
                for q in range(NQ):
                    io[q][sl] = zi
                    do[q][sl] = pr
                    wo[q][sl] = zf
                return 0

            lax.fori_loop(0, CAPQ // 16, fill, 0)

            def part(g, pos):
                sl = pl.ds(g * 16, 16)
                s = idx_v[sl]
                d = dst_v[sl]
                w = w_v[sl]
                newpos = []
                for q in range(NQ):
                    mq = (d >= q * VQ) & (d < (q + 1) * VQ)
                    pq = pos[q]
                    plsc.store_compressed(io[q].at[pl.ds(pq, 16)], s, mask=mq)
                    plsc.store_compressed(do[q].at[pl.ds(pq, 16)], d - q * VQ,
                                          mask=mq)
                    plsc.store_compressed(wo[q].at[pl.ds(pq, 16)], w, mask=mq)
                    newpos.append(pq + jnp.sum(mq.astype(i32)))
                return tuple(newpos)

            lax.fori_loop(0, EPT // 16, part, (0,) * NQ)

            for q in range(NQ):
                pltpu.sync_copy(io[q].at[pl.ds(0, CAPP)], pidx.at[sid, q, 0])
                pltpu.sync_copy(do[q].at[pl.ds(0, CAPP)], pdst.at[sid, q, 0])
                pltpu.sync_copy(wo[q].at[pl.ds(0, CAPP)], pw.at[sid, q, 0])

    return prep


def _make_sc_spmm(V, E):
    NCH = CAPP // NE
    NOUT = NCH // RING
    RPT = VQP // NT
    mesh = plsc.VectorSubcoreMesh(core_axis_name="c", subcore_axis_name="s")

    @functools.partial(
        pl.kernel,
        out_type=jax.ShapeDtypeStruct((4, NQ, VQP, F), jnp.float32),
        mesh=mesh,
        compiler_params=pltpu.CompilerParams(needs_layout_passes=False),
        scratch_types=[
            pltpu.VMEM((CAPP,), jnp.int32),
            pltpu.VMEM((CAPP,), jnp.float32),
            pltpu.VMEM((CAPP,), jnp.int32),
            pltpu.VMEM((RING, NE), jnp.int32),
            [pltpu.VMEM((NE, F), jnp.float32) for _ in range(RING)],
            [pltpu.VMEM((NE, F), jnp.float32) for _ in range(RING)],
            pltpu.VMEM_SHARED((VQP, F), jnp.float32),
            pltpu.SemaphoreType.DMA((RING,)),
            pltpu.SemaphoreType.DMA((RING,)),
        ],
    )
    def spmm(tbl, pidx, pdst, pw, zeros_h, out, idx_v, w_v, dst_v, dstc,
             grows, srows, acc, gsem, ssem):
        cid = lax.axis_index("c")
        sid = lax.axis_index("s")

        def gather(c, b):
            pltpu.async_copy(tbl.at[idx_v.at[pl.ds(c * NE, NE)]], grows[b],
                             gsem.at[b])

        def gather_wait(b):
            pltpu.make_async_copy(tbl.at[idx_v.at[pl.ds(0, NE)]], grows[b],
                                  gsem.at[b]).wait()

        def scatter(c, b):
            for q in range(NE // 16):
                dstc[b, pl.ds(q * 16, 16)] = dst_v[pl.ds(c * NE + q * 16, 16)]
            pltpu.async_copy(srows[b], acc.at[dstc.at[b]], ssem.at[b],
                             add=True)

        def scatter_wait(b):
            pltpu.make_async_copy(srows[b], acc.at[dstc.at[0]],
                                  ssem.at[b]).wait()

        def scale(c, b):
            for g in range(NE // 16):
                wg = w_v[pl.ds(c * NE + g * 16, 16)]
                for i in range(16):
                    bc = lax.gather(
                        wg, jnp.full((16, 1), i, jnp.int32),
                        lax.GatherDimensionNumbers(
                            offset_dims=(), collapsed_slice_dims=(0,),
                            start_index_map=(0,)),
                        slice_sizes=(1,),
                        mode=lax.GatherScatterMode.PROMISE_IN_BOUNDS)
                    e = g * 16 + i
                    for j in range(F // 16):
                        sl = pl.ds(j * 16, 16)
                        srows[b][e, sl] = grows[b][e, sl] * bc

        def quarter_body(qq, _):
            pltpu.sync_copy(pidx.at[sid, qq, 0], idx_v)
            pltpu.sync_copy(pw.at[sid, qq, 0], w_v)
            pltpu.sync_copy(pdst.at[sid, qq, 0], dst_v)

            def batch_body(k, __):
                bidx = cid * 2 + k
                off = jnp.where(k == 0, cid * 2 * V, V)

                def add_off(o, ___):
                    for u in range(4):
                        s = o * 64 + u * 16
                        idx_v[pl.ds(s, 16)] = idx_v[pl.ds(s, 16)] + off
                    return 0

                lax.fori_loop(0, CAPP // 64, add_off, 0)

                pltpu.sync_copy(zeros_h, acc.at[pl.ds(sid * RPT, RPT)])
                plsc.subcore_barrier()

                for b in range(RING):
                    gather(b, b)

                def outer(o, __):
                    for b in range(RING):
                        c = o * RING + b
                        gather_wait(b)

                        @pl.when(o >= 1)
                        def drain_prev():
                            scatter_wait(b)

                        scale(c, b)
                        scatter(c, b)

                        @pl.when(o < NOUT - 1)
                        def refill():
                            gather((o + 1) * RING + b, b)
                    return 0

                lax.fori_loop(0, NOUT, outer, 0)
                for b in range(RING):
                    scatter_wait(b)
                plsc.subcore_barrier()
                pltpu.sync_copy(acc.at[pl.ds(sid * RPT, RPT)],
                                out.at[bidx, qq, pl.ds(sid * RPT, RPT)])
                plsc.subcore_barrier()
                return 0

            lax.fori_loop(0, 2, batch_body, 0)
            return 0

        lax.fori_loop(0, NQ, quarter_body, 0)

    return spmm


_BLK = 4000


def _k_up_matmul(x2d, wcat, n_rows_out):
    n = wcat.shape[1]
    grid = n_rows_out // _BLK

    def body(x_ref, w_ref, o_ref):
        xb = jnp.repeat(x_ref[...], 4, axis=0)
        o_ref[...] = jnp.dot(xb, w_ref[...],
                             preferred_element_type=jnp.float32)

    return pl.pallas_call(
        body,
        grid=(grid,),
        in_specs=[
            pl.BlockSpec((_BLK // 4, F), lambda g: (g, 0)),
            pl.BlockSpec(wcat.shape, lambda g: (0, 0)),
        ],
        out_specs=pl.BlockSpec((_BLK, n), lambda g: (g, 0)),
        out_shape=jax.ShapeDtypeStruct((n_rows_out, n), jnp.float32),
    )(x2d, wcat)


def _k_axpy2(a2d, s2d):
    R = a2d.shape[0]

    def body(a_ref, s_ref, o_ref):
        o_ref[...] = a_ref[...] + 2.0 * s_ref[...]

    return pl.pallas_call(
        body,
        grid=(R // _BLK,),
        in_specs=[pl.BlockSpec((_BLK, F), lambda g: (g, 0))] * 2,
        out_specs=pl.BlockSpec((_BLK, F), lambda g: (g, 0)),
        out_shape=jax.ShapeDtypeStruct((R, F), jnp.float32),
    )(a2d, s2d)


def _k_add_stats(base2d, s2d):
    R = base2d.shape[0]

    def body(b_ref, s_ref, o_ref, st_ref):
        t = b_ref[...] + s_ref[...]
        o_ref[...] = t

        @pl.when(pl.program_id(0) == 0)
        def _():
            st_ref[...] = jnp.zeros_like(st_ref)

        st_ref[pl.ds(0, 1), :] += jnp.sum(t, axis=0, keepdims=True)
        st_ref[pl.ds(1, 1), :] += jnp.sum(t * t, axis=0, keepdims=True)

    return pl.pallas_call(
        body,
        grid=(R // _BLK,),
        in_specs=[pl.BlockSpec((_BLK, F), lambda g: (g, 0))] * 2,
        out_specs=[
            pl.BlockSpec((_BLK, F), lambda g: (g, 0)),
            pl.BlockSpec((8, F), lambda g: (0, 0)),
        ],
        out_shape=[
            jax.ShapeDtypeStruct((R, F), jnp.float32),
            jax.ShapeDtypeStruct((8, F), jnp.float32),
        ],
    )(base2d, s2d)


def _bn_coeffs(st_ref, g_ref, bt_ref, n_rows):
    mu = st_ref[pl.ds(0, 1), :] / n_rows
    var = st_ref[pl.ds(1, 1), :] / n_rows - mu * mu
    inv = lax.rsqrt(var + 1e-5) * g_ref[pl.ds(0, 1), :]
    return inv, bt_ref[pl.ds(0, 1), :] - mu * inv


def _k_bn_dualmm(h2d, st, skip2d, whcat, wscat, g, bt):
    R = h2d.shape[0]
    n = whcat.shape[1]

    def body(h_ref, st_ref, sk_ref, wh_ref, ws_ref, g_ref, bt_ref, o_ref):
        inv, sh = _bn_coeffs(st_ref, g_ref, bt_ref, float(R))
        hb = jax.nn.relu(h_ref[...] * inv + sh)
        o_ref[...] = (
            jnp.dot(hb, wh_ref[...], preferred_element_type=jnp.float32)
            + jnp.dot(sk_ref[...], ws_ref[...],
                      preferred_element_type=jnp.float32))

    return pl.pallas_call(
        body,
        grid=(R // _BLK,),
        in_specs=[
            pl.BlockSpec((_BLK, F), lambda g_: (g_, 0)),
            pl.BlockSpec((8, F), lambda g_: (0, 0)),
            pl.BlockSpec((_BLK, F), lambda g_: (g_, 0)),
            pl.BlockSpec((F, n), lambda g_: (0, 0)),
            pl.BlockSpec((F, n), lambda g_: (0, 0)),
            pl.BlockSpec((1, F), lambda g_: (0, 0)),
            pl.BlockSpec((1, F), lambda g_: (0, 0)),
        ],
        out_specs=pl.BlockSpec((_BLK, n), lambda g_: (g_, 0)),
        out_shape=jax.ShapeDtypeStruct((R, n), jnp.float32),
    )(h2d, st, skip2d, whcat, wscat, g, bt)


def _k_bn_relu(h2d, st, g, bt):
    R = h2d.shape[0]

    def body(h_ref, st_ref, g_ref, bt_ref, o_ref):
        inv, sh = _bn_coeffs(st_ref, g_ref, bt_ref, float(R))
        o_ref[...] = jax.nn.relu(h_ref[...] * inv + sh)

    return pl.pallas_call(
        body,
        grid=(R // _BLK,),
        in_specs=[
            pl.BlockSpec((_BLK, F), lambda g_: (g_, 0)),
            pl.BlockSpec((8, F), lambda g_: (0, 0)),
            pl.BlockSpec((1, F), lambda g_: (0, 0)),
            pl.BlockSpec((1, F), lambda g_: (0, 0)),
        ],
        out_specs=pl.BlockSpec((_BLK, F), lambda g_: (g_, 0)),
        out_shape=jax.ShapeDtypeStruct((R, F), jnp.float32),
    )(h2d, st, g, bt)


def kernel(x, concat_data, edge_index, edge_weight, w1, b1, g1, bt1,
           w2, b2, g2, bt2):
    B, V_in, _ = x.shape
    V = concat_data.shape[1]
    E = edge_weight.shape[0]
    R = B * V

    dst = edge_index[0]
    src = edge_index[1]

    prep = _make_sc_prep(V, E)
    pidx, pdst, pw = prep(src, dst, edge_weight)

    spmm_p = _make_sc_spmm(V, E)
    zeros_h = jnp.zeros((VQP // NT, F), jnp.float32)

    def spmm(tbl):
        s = spmm_p(tbl, pidx, pdst, pw, zeros_h)
        return s[:, :, :VQ, :].reshape(R, F)

    wcat1 = jnp.concatenate([w1[1], w1[2], w1[0] - w1[2]], axis=1)
    apb1 = _k_up_matmul(x.reshape(B * V_in, F), wcat1, R)
    a1 = apb1[:, :F]
    p1 = apb1[:, F:2 * F]
    base1 = apb1[:, 2 * F:]
    s2 = spmm(p1)
    q1 = _k_axpy2(a1, s2)
    s3 = spmm(q1)
    h1, st1 = _k_add_stats(base1, s3)

    wh = jnp.concatenate([w2[1, :F], w2[2, :F], (w2[0] - w2[2])[:F]], axis=1)
    ws = jnp.concatenate([w2[1, F:], w2[2, F:], (w2[0] - w2[2])[F:]], axis=1)
    skip2d = concat_data.reshape(R, F)
    apb2 = _k_bn_dualmm(h1, st1, skip2d, wh, ws,
                        g1.reshape(1, F), bt1.reshape(1, F))
    a2 = apb2[:, :F]
    p2 = apb2[:, F:2 * F]
    base2 = apb2[:, 2 * F:]
    s2b = spmm(p2)
    q2 = _k_axpy2(a2, s2b)
    s3b = spmm(q2)
    h2, st2 = _k_add_stats(base2, s3b)
    out = _k_bn_relu(h2, st2, g2.reshape(1, F), bt2.reshape(1, F))
    return out.reshape(B, V, F)
